# baseline (speedup 1.0000x reference)
import functools

import jax
import jax.numpy as jnp
from jax import lax
from jax.experimental import pallas as pl
from jax.experimental.pallas import tpu as pltpu
from jax.experimental.pallas import tpu_sc as plsc

N = 10000
E = 320000
D_IN = 128
D_OUT = 300
G = 16

NC = 2
NS = 16
NW = NC * NS
EPT = E // NW
CH = 125
NCH = EPT // CH
NPH = 16
NP = NCH // NPH
NCHD = (E // NS) // CH
RSTEP = 624
RSPAN = 640
DEGW = 16
YCH = 80

@functools.lru_cache(maxsize=None)
def _sc_kernels():
    mesh = plsc.VectorSubcoreMesh(core_axis_name="c", subcore_axis_name="s",
                                  num_cores=NC, num_subcores=NS)

    @functools.partial(
        pl.kernel,
        out_type=jax.ShapeDtypeStruct((N, DEGW), jnp.float32),
        mesh=mesh,
        scratch_types=[
            pltpu.VMEM((NCHD, CH), jnp.int32),
            pltpu.VMEM((CH, DEGW), jnp.float32),
            pltpu.VMEM_SHARED((N, DEGW), jnp.float32),
            pltpu.SemaphoreType.DMA,
        ],
    )
    def _deg_kernel(dst_hbm, ones_hbm, one_c_hbm, out_hbm, dst_v, ones_v,
                    deg_sh, sem):
        s = lax.axis_index("s")
        r0 = s * RSTEP
        pltpu.sync_copy(one_c_hbm.at[pl.ds(r0, RSPAN)],
                        deg_sh.at[pl.ds(r0, RSPAN)])
        pltpu.sync_copy(dst_hbm.at[s], dst_v)
        pltpu.sync_copy(ones_hbm, ones_v)
        plsc.subcore_barrier()

        def fire(j, carry):
            pltpu.async_copy(ones_v, deg_sh.at[dst_v.at[j]], sem, add=True)
            return carry

        lax.fori_loop(0, NCHD, fire, 0)

        def drain(j, carry):
            pltpu.make_async_copy(ones_v, deg_sh.at[dst_v.at[j]], sem).wait()
            return carry

        lax.fori_loop(0, NCHD, drain, 0)
        plsc.subcore_barrier()
        pltpu.sync_copy(deg_sh.at[pl.ds(r0, RSPAN)],
                        out_hbm.at[pl.ds(r0, RSPAN)])

    @functools.partial(
        pl.kernel,
        out_type=(jax.ShapeDtypeStruct((NC, N, D_IN), jnp.float32),
                  jax.ShapeDtypeStruct((N, D_IN), jnp.float32)),
        mesh=mesh,
        scratch_types=[
            pltpu.VMEM((NPH, CH), jnp.int32),
            pltpu.VMEM((NPH, CH), jnp.int32),
            pltpu.VMEM((CH, D_IN), jnp.float32),
            pltpu.VMEM((CH, D_IN), jnp.float32),
            pltpu.VMEM((YCH, DEGW), jnp.float32),
            pltpu.VMEM_SHARED((N, D_IN), jnp.float32),
            pltpu.SemaphoreType.DMA,
            pltpu.SemaphoreType.DMA,
        ],
    )
    def _scatter_kernel(src_hbm, dst_hbm, deg_hbm, x_hbm,
                        out_hbm, u_hbm,
                        src_v, dst_v, buf_a, buf_b, degb, s_sh,
                        sem_a, sem_b):
        c = lax.axis_index("c")
        s = lax.axis_index("s")
        w = s * NC + c
        r0 = s * RSTEP

        def urow(r, carry):
            d = degb[r, :]
            ib = lax.bitcast_convert_type(d, jnp.int32)
            ib = jnp.int32(0x5F3759DF) - lax.shift_right_logical(ib, 1)
            z = lax.bitcast_convert_type(ib, jnp.float32)
            hd = d * 0.5
            z = z * (1.5 - hd * z * z)
            z = z * (1.5 - hd * z * z)
            z = z * (1.5 - hd * z * z)
            z = z * 0.5
            for q in range(D_IN // 16):
                sl = pl.ds(16 * q, 16)
                buf_a[r, sl] = buf_a[r, sl] * z
            return carry

        for k in range(RSPAN // YCH):
            rr = r0 + YCH * k
            pltpu.sync_copy(deg_hbm.at[pl.ds(rr, YCH)], degb)
            pltpu.sync_copy(x_hbm.at[pl.ds(rr, YCH)], buf_a.at[pl.ds(0, YCH)])
            lax.fori_loop(0, YCH, urow, 0)
            pltpu.sync_copy(buf_a.at[pl.ds(0, YCH)], u_hbm.at[pl.ds(rr, YCH)])
            pltpu.sync_copy(buf_a.at[pl.ds(0, YCH)], s_sh.at[pl.ds(rr, YCH)])
        plsc.subcore_barrier()

        def body(jj, carry):
            j0 = 2 * jj
            pltpu.async_copy(u_hbm.at[src_v.at[j0 + 1]], buf_b, sem_b)
            pltpu.make_async_copy(u_hbm.at[src_v.at[j0]], buf_a, sem_a).wait()
            pltpu.sync_copy(buf_a, s_sh.at[dst_v.at[j0]], add=True)
            jn = jnp.minimum(j0 + 2, NPH - 1)
            pltpu.async_copy(u_hbm.at[src_v.at[jn]], buf_a, sem_a)
            pltpu.make_async_copy(u_hbm.at[src_v.at[j0 + 1]], buf_b,
                                  sem_b).wait()
            pltpu.sync_copy(buf_b, s_sh.at[dst_v.at[j0 + 1]], add=True)
            return carry

        for ph in range(NP):
            pltpu.sync_copy(src_hbm.at[w].at[pl.ds(ph * NPH, NPH)], src_v)
            pltpu.sync_copy(dst_hbm.at[w].at[pl.ds(ph * NPH, NPH)], dst_v)
            pltpu.async_copy(u_hbm.at[src_v.at[0]], buf_a, sem_a)
            lax.fori_loop(0, NPH // 2, body, 0)
            pltpu.make_async_copy(u_hbm.at[src_v.at[NPH - 1]], buf_a,
                                  sem_a).wait()
        plsc.subcore_barrier()
        pltpu.sync_copy(s_sh.at[pl.ds(r0, RSPAN)],
                        out_hbm.at[c].at[pl.ds(r0, RSPAN)])

    return _deg_kernel, _scatter_kernel


_RB = 1000
_NB = N // _RB


def _final_body(deg_ref, s0_ref, s1_ref, x_ref, batch_ref, w_ref,
                b_ref, h_ref, ge_ref, cnt_scr):
    i = pl.program_id(0)
    dinv = lax.rsqrt(deg_ref[:, 0:1])
    agg = ((s0_ref[...] + s1_ref[...]) * (2.0 * dinv)
           - x_ref[...] * (dinv * dinv))
    h = jnp.maximum(
        lax.dot_general(agg, w_ref[...], (((1,), (0,)), ((), ())),
                        precision=lax.Precision.HIGHEST,
                        preferred_element_type=jnp.float32) + b_ref[...],
        0.0)
    h_ref[...] = h

    oh = (lax.broadcasted_iota(jnp.int32, (G, _RB), 0)
          == batch_ref[0, 0:1, :]).astype(jnp.float32)
    part = lax.dot_general(oh, h, (((1,), (0,)), ((), ())),
                           precision=lax.Precision.HIGHEST,
                           preferred_element_type=jnp.float32)
    cpart = jnp.broadcast_to(jnp.sum(oh, axis=1, keepdims=True), (G, 128))

    @pl.when(i == 0)
    def _init():
        ge_ref[...] = jnp.zeros_like(ge_ref)
        cnt_scr[...] = jnp.zeros_like(cnt_scr)

    ge_ref[...] += part
    cnt_scr[...] += cpart

    @pl.when(i == _NB - 1)
    def _fin():
        ge_ref[...] = ge_ref[...] / jnp.maximum(cnt_scr[:, 0:1], 1.0)


_final_kernel = pl.pallas_call(
    _final_body,
    grid=(_NB,),
    in_specs=[
        pl.BlockSpec((_RB, DEGW), lambda i: (i, 0)),
        pl.BlockSpec((_RB, D_IN), lambda i: (i, 0)),
        pl.BlockSpec((_RB, D_IN), lambda i: (i, 0)),
        pl.BlockSpec((_RB, D_IN), lambda i: (i, 0)),
        pl.BlockSpec((1, 1, _RB), lambda i: (i, 0, 0)),
        pl.BlockSpec((D_IN, D_OUT), lambda i: (0, 0)),
        pl.BlockSpec((1, D_OUT), lambda i: (0, 0)),
    ],
    out_specs=[
        pl.BlockSpec((_RB, D_OUT), lambda i: (i, 0)),
        pl.BlockSpec((G, D_OUT), lambda i: (0, 0)),
    ],
    out_shape=[
        jax.ShapeDtypeStruct((N, D_OUT), jnp.float32),
        jax.ShapeDtypeStruct((G, D_OUT), jnp.float32),
    ],
    scratch_shapes=[pltpu.VMEM((G, 128), jnp.float32)],
)


def kernel(x, edge_index, batch, W, b):
    src3 = edge_index[0].reshape(NW, NCH, CH)
    dst3 = edge_index[1].reshape(NW, NCH, CH)
    dst16 = edge_index[1].reshape(NS, NCHD, CH)
    ones_c = jnp.ones((CH, DEGW), jnp.float32)
    one_c = jnp.ones((N, DEGW), jnp.float32)
    deg_kernel, scatter_kernel = _sc_kernels()
    deg = deg_kernel(dst16, ones_c, one_c)
    sp, _u = scatter_kernel(src3, dst3, deg, x)
    h, ge = _final_kernel(deg, sp[0], sp[1], x,
                          batch.reshape(_NB, 1, _RB), W, b.reshape(1, D_OUT))
    return (h, ge)


# speedup vs baseline: 1.1160x; 1.1160x over previous
import functools

import jax
import jax.numpy as jnp
from jax import lax
from jax.experimental import pallas as pl
from jax.experimental.pallas import tpu as pltpu
from jax.experimental.pallas import tpu_sc as plsc

N = 10000
E = 320000
D_IN = 128
D_OUT = 300
G = 16

NC = 2
NS = 16
NW = NC * NS
EPT = E // NW
CH = 125
NCH = EPT // CH
NPH = 16
NP = NCH // NPH
RSTEP = 624
RSPAN = 640
DEGW = 16


@functools.lru_cache(maxsize=None)
def _sc_kernels():
    mesh = plsc.VectorSubcoreMesh(core_axis_name="c", subcore_axis_name="s",
                                  num_cores=NC, num_subcores=NS)

    @functools.partial(
        pl.kernel,
        out_type=jax.ShapeDtypeStruct((NC, N, DEGW), jnp.float32),
        mesh=mesh,
        scratch_types=[
            pltpu.VMEM((NCH, CH), jnp.int32),
            pltpu.VMEM((CH, DEGW), jnp.float32),
            pltpu.VMEM_SHARED((N, DEGW), jnp.float32),
            pltpu.SemaphoreType.DMA,
        ],
    )
    def _deg_kernel(dst_hbm, ones_hbm, half_hbm, out_hbm, dst_v, ones_v,
                    deg_sh, sem):
        c = lax.axis_index("c")
        s = lax.axis_index("s")
        w = s * NC + c
        r0 = s * RSTEP
        pltpu.sync_copy(half_hbm.at[pl.ds(r0, RSPAN)],
                        deg_sh.at[pl.ds(r0, RSPAN)])
        pltpu.sync_copy(dst_hbm.at[w], dst_v)
        pltpu.sync_copy(ones_hbm, ones_v)
        plsc.subcore_barrier()

        def fire(j, carry):
            pltpu.async_copy(ones_v, deg_sh.at[dst_v.at[j]], sem, add=True)
            return carry

        lax.fori_loop(0, NCH, fire, 0)

        def drain(j, carry):
            pltpu.make_async_copy(ones_v, deg_sh.at[dst_v.at[j]], sem).wait()
            return carry

        lax.fori_loop(0, NCH, drain, 0)
        plsc.subcore_barrier()
        pltpu.sync_copy(deg_sh.at[pl.ds(r0, RSPAN)],
                        out_hbm.at[c].at[pl.ds(r0, RSPAN)])

    @functools.partial(
        pl.kernel,
        out_type=jax.ShapeDtypeStruct((NC, N, D_IN), jnp.float32),
        mesh=mesh,
        scratch_types=[
            pltpu.VMEM((NCH, CH), jnp.int32),
            pltpu.VMEM((2, NPH, CH), jnp.int32),
            pltpu.VMEM((CH, D_IN), jnp.float32),
            pltpu.VMEM((CH, D_IN), jnp.float32),
            pltpu.VMEM_SHARED((N, D_IN), jnp.float32),
            pltpu.SemaphoreType.DMA,
            pltpu.SemaphoreType.DMA,
            pltpu.SemaphoreType.DMA,
        ],
    )
    def _scatter_kernel(src_hbm, dst_hbm, y_hbm, yh_hbm, out_hbm,
                        src_v, dst_v, buf_a, buf_b, s_sh,
                        sem_a, sem_b, sem_d):
        c = lax.axis_index("c")
        s = lax.axis_index("s")
        w = s * NC + c
        r0 = s * RSTEP
        pltpu.sync_copy(src_hbm.at[w], src_v)
        pltpu.sync_copy(dst_hbm.at[w].at[pl.ds(0, NPH)], dst_v.at[0])
        pltpu.async_copy(dst_hbm.at[w].at[pl.ds(NPH, NPH)], dst_v.at[1],
                         sem_d)
        pltpu.sync_copy(yh_hbm.at[pl.ds(r0, RSPAN)], s_sh.at[pl.ds(r0, RSPAN)])
        plsc.subcore_barrier()

        pltpu.async_copy(y_hbm.at[src_v.at[0]], buf_a, sem_a)

        def body(jj, carry):
            j0 = 2 * jj
            j1 = j0 + 1
            p = j0 // NPH
            pb = lax.rem(p, 2)

            @pl.when(jnp.logical_and(lax.rem(j0, NPH) == 0, j0 > 0))
            def _phase():
                pltpu.make_async_copy(
                    dst_hbm.at[w].at[pl.ds(p * NPH, NPH)], dst_v.at[pb],
                    sem_d).wait()

                @pl.when(p + 1 < NP)
                def _pref():
                    pltpu.async_copy(
                        dst_hbm.at[w].at[pl.ds((p + 1) * NPH, NPH)],
                        dst_v.at[lax.rem(p + 1, 2)], sem_d)

            pltpu.async_copy(y_hbm.at[src_v.at[j1]], buf_b, sem_b)
            pltpu.make_async_copy(y_hbm.at[src_v.at[j0]], buf_a, sem_a).wait()
            pltpu.sync_copy(buf_a, s_sh.at[dst_v.at[pb, lax.rem(j0, NPH)]],
                            add=True)
            jn = jnp.minimum(j0 + 2, NCH - 1)
            pltpu.async_copy(y_hbm.at[src_v.at[jn]], buf_a, sem_a)
            pltpu.make_async_copy(y_hbm.at[src_v.at[j1]], buf_b, sem_b).wait()
            pltpu.sync_copy(buf_b, s_sh.at[dst_v.at[pb, lax.rem(j1, NPH)]],
                            add=True)
            return carry

        lax.fori_loop(0, NCH // 2, body, 0)
        pltpu.make_async_copy(y_hbm.at[src_v.at[NCH - 1]], buf_a, sem_a).wait()
        plsc.subcore_barrier()
        pltpu.sync_copy(s_sh.at[pl.ds(r0, RSPAN)],
                        out_hbm.at[c].at[pl.ds(r0, RSPAN)])

    return _deg_kernel, _scatter_kernel


_RB_Y = 1000


def _y_body(d0_ref, d1_ref, x_ref, y_ref, yh_ref):
    deg = d0_ref[:, 0:1] + d1_ref[:, 0:1]
    dinv = lax.rsqrt(deg)
    y = x_ref[...] * dinv
    y_ref[...] = y
    yh_ref[...] = y * 0.5


_y_kernel = pl.pallas_call(
    _y_body,
    grid=(N // _RB_Y,),
    in_specs=[
        pl.BlockSpec((_RB_Y, DEGW), lambda i: (i, 0)),
        pl.BlockSpec((_RB_Y, DEGW), lambda i: (i, 0)),
        pl.BlockSpec((_RB_Y, D_IN), lambda i: (i, 0)),
    ],
    out_specs=[
        pl.BlockSpec((_RB_Y, D_IN), lambda i: (i, 0)),
        pl.BlockSpec((_RB_Y, D_IN), lambda i: (i, 0)),
    ],
    out_shape=[
        jax.ShapeDtypeStruct((N, D_IN), jnp.float32),
        jax.ShapeDtypeStruct((N, D_IN), jnp.float32),
    ],
)


_RB = 1000
_NB = N // _RB


def _final_body(d0_ref, d1_ref, s0_ref, s1_ref, batch_ref, w_ref, b_ref,
                h_ref, ge_ref, cnt_scr):
    i = pl.program_id(0)
    deg = d0_ref[:, 0:1] + d1_ref[:, 0:1]
    dinv = lax.rsqrt(deg)
    agg = (s0_ref[...] + s1_ref[...]) * dinv
    h = jnp.maximum(
        lax.dot_general(agg, w_ref[...], (((1,), (0,)), ((), ())),
                        precision=lax.Precision.HIGHEST,
                        preferred_element_type=jnp.float32) + b_ref[...],
        0.0)
    h_ref[...] = h

    oh = (lax.broadcasted_iota(jnp.int32, (G, _RB), 0)
          == batch_ref[0, 0:1, :]).astype(jnp.float32)
    part = lax.dot_general(oh, h, (((1,), (0,)), ((), ())),
                           precision=lax.Precision.HIGHEST,
                           preferred_element_type=jnp.float32)
    cpart = jnp.broadcast_to(jnp.sum(oh, axis=1, keepdims=True), (G, 128))

    @pl.when(i == 0)
    def _init():
        ge_ref[...] = jnp.zeros_like(ge_ref)
        cnt_scr[...] = jnp.zeros_like(cnt_scr)

    ge_ref[...] += part
    cnt_scr[...] += cpart

    @pl.when(i == _NB - 1)
    def _fin():
        ge_ref[...] = ge_ref[...] / jnp.maximum(cnt_scr[:, 0:1], 1.0)


_final_kernel = pl.pallas_call(
    _final_body,
    grid=(_NB,),
    in_specs=[
        pl.BlockSpec((_RB, DEGW), lambda i: (i, 0)),
        pl.BlockSpec((_RB, DEGW), lambda i: (i, 0)),
        pl.BlockSpec((_RB, D_IN), lambda i: (i, 0)),
        pl.BlockSpec((_RB, D_IN), lambda i: (i, 0)),
        pl.BlockSpec((1, 1, _RB), lambda i: (i, 0, 0)),
        pl.BlockSpec((D_IN, D_OUT), lambda i: (0, 0)),
        pl.BlockSpec((1, D_OUT), lambda i: (0, 0)),
    ],
    out_specs=[
        pl.BlockSpec((_RB, D_OUT), lambda i: (i, 0)),
        pl.BlockSpec((G, D_OUT), lambda i: (0, 0)),
    ],
    out_shape=[
        jax.ShapeDtypeStruct((N, D_OUT), jnp.float32),
        jax.ShapeDtypeStruct((G, D_OUT), jnp.float32),
    ],
    scratch_shapes=[pltpu.VMEM((G, 128), jnp.float32)],
)


def kernel(x, edge_index, batch, W, b):
    src3 = edge_index[0].reshape(NW, NCH, CH)
    dst3 = edge_index[1].reshape(NW, NCH, CH)
    ones_c = jnp.ones((CH, DEGW), jnp.float32)
    half_c = jnp.full((N, DEGW), 0.5, jnp.float32)
    deg_kernel, scatter_kernel = _sc_kernels()
    degp = deg_kernel(dst3, ones_c, half_c)
    y, yh = _y_kernel(degp[0], degp[1], x)
    sp = scatter_kernel(src3, dst3, y, yh)
    h, ge = _final_kernel(degp[0], degp[1], sp[0], sp[1],
                          batch.reshape(_NB, 1, _RB), W, b.reshape(1, D_OUT))
    return (h, ge)


# speedup vs baseline: 1.2239x; 1.0967x over previous
import functools

import jax
import jax.numpy as jnp
from jax import lax
from jax.experimental import pallas as pl
from jax.experimental.pallas import tpu as pltpu
from jax.experimental.pallas import tpu_sc as plsc

N = 10000
E = 320000
D_IN = 128
D_OUT = 300
G = 16

NC = 2
NS = 16
NW = NC * NS
EPT = E // NW
CH = 125
NCH = EPT // CH
NPH = 16
NP = NCH // NPH
RSTEP = 624
RSPAN = 640
DEGW = 16


@functools.lru_cache(maxsize=None)
def _sc_kernels():
    mesh = plsc.VectorSubcoreMesh(core_axis_name="c", subcore_axis_name="s",
                                  num_cores=NC, num_subcores=NS)

    @functools.partial(
        pl.kernel,
        out_type=jax.ShapeDtypeStruct((NC, N, DEGW), jnp.float32),
        mesh=mesh,
        scratch_types=[
            pltpu.VMEM((NCH, CH), jnp.int32),
            pltpu.VMEM((CH, DEGW), jnp.float32),
            pltpu.VMEM((40, DEGW), jnp.float32),
            pltpu.VMEM_SHARED((N, DEGW), jnp.float32),
            pltpu.SemaphoreType.DMA,
        ],
    )
    def _deg_kernel(dst_hbm, out_hbm, dst_v, ones_v, half_v, deg_sh, sem):
        c = lax.axis_index("c")
        s = lax.axis_index("s")
        w = s * NC + c
        r0 = s * RSTEP
        pltpu.sync_copy(dst_hbm.at[w], dst_v)

        def fill_ones(i, carry):
            ones_v[i, :] = jnp.full((DEGW,), 1.0, jnp.float32)
            return carry

        lax.fori_loop(0, CH, fill_ones, 0)

        def fill_half(i, carry):
            half_v[i, :] = jnp.full((DEGW,), 0.5, jnp.float32)
            return carry

        lax.fori_loop(0, 40, fill_half, 0)
        for k in range(RSPAN // 40):
            pltpu.sync_copy(half_v, deg_sh.at[pl.ds(r0 + 40 * k, 40)])
        plsc.subcore_barrier()

        def fire(j, carry):
            pltpu.async_copy(ones_v, deg_sh.at[dst_v.at[j]], sem, add=True)
            return carry

        lax.fori_loop(0, NCH, fire, 0)

        def drain(j, carry):
            pltpu.make_async_copy(ones_v, deg_sh.at[dst_v.at[j]], sem).wait()
            return carry

        lax.fori_loop(0, NCH, drain, 0)
        plsc.subcore_barrier()
        pltpu.sync_copy(deg_sh.at[pl.ds(r0, RSPAN)],
                        out_hbm.at[c].at[pl.ds(r0, RSPAN)])

    @functools.partial(
        pl.kernel,
        out_type=jax.ShapeDtypeStruct((NC, N, D_IN), jnp.float32),
        mesh=mesh,
        scratch_types=[
            pltpu.VMEM((NCH, CH), jnp.int32),
            pltpu.VMEM((2, NPH, CH), jnp.int32),
            pltpu.VMEM((CH, D_IN), jnp.float32),
            pltpu.VMEM((CH, D_IN), jnp.float32),
            pltpu.VMEM_SHARED((N, D_IN), jnp.float32),
            pltpu.SemaphoreType.DMA,
            pltpu.SemaphoreType.DMA,
            pltpu.SemaphoreType.DMA,
        ],
    )
    def _scatter_kernel(src_hbm, dst_hbm, y_hbm, yh_hbm, out_hbm,
                        src_v, dst_v, buf_a, buf_b, s_sh,
                        sem_a, sem_b, sem_d):
        c = lax.axis_index("c")
        s = lax.axis_index("s")
        w = s * NC + c
        r0 = s * RSTEP
        pltpu.sync_copy(src_hbm.at[w], src_v)
        pltpu.sync_copy(dst_hbm.at[w].at[pl.ds(0, NPH)], dst_v.at[0])
        pltpu.async_copy(dst_hbm.at[w].at[pl.ds(NPH, NPH)], dst_v.at[1],
                         sem_d)
        pltpu.sync_copy(yh_hbm.at[pl.ds(r0, RSPAN)], s_sh.at[pl.ds(r0, RSPAN)])
        plsc.subcore_barrier()

        pltpu.async_copy(y_hbm.at[src_v.at[0]], buf_a, sem_a)

        def body(jj, carry):
            j0 = 2 * jj
            j1 = j0 + 1
            p = j0 // NPH
            pb = lax.rem(p, 2)

            @pl.when(jnp.logical_and(lax.rem(j0, NPH) == 0, j0 > 0))
            def _phase():
                pltpu.make_async_copy(
                    dst_hbm.at[w].at[pl.ds(p * NPH, NPH)], dst_v.at[pb],
                    sem_d).wait()

                @pl.when(p + 1 < NP)
                def _pref():
                    pltpu.async_copy(
                        dst_hbm.at[w].at[pl.ds((p + 1) * NPH, NPH)],
                        dst_v.at[lax.rem(p + 1, 2)], sem_d)

            pltpu.async_copy(y_hbm.at[src_v.at[j1]], buf_b, sem_b)
            pltpu.make_async_copy(y_hbm.at[src_v.at[j0]], buf_a, sem_a).wait()
            pltpu.sync_copy(buf_a, s_sh.at[dst_v.at[pb, lax.rem(j0, NPH)]],
                            add=True)
            jn = jnp.minimum(j0 + 2, NCH - 1)
            pltpu.async_copy(y_hbm.at[src_v.at[jn]], buf_a, sem_a)
            pltpu.make_async_copy(y_hbm.at[src_v.at[j1]], buf_b, sem_b).wait()
            pltpu.sync_copy(buf_b, s_sh.at[dst_v.at[pb, lax.rem(j1, NPH)]],
                            add=True)
            return carry

        lax.fori_loop(0, NCH // 2, body, 0)
        pltpu.make_async_copy(y_hbm.at[src_v.at[NCH - 1]], buf_a, sem_a).wait()
        plsc.subcore_barrier()
        pltpu.sync_copy(s_sh.at[pl.ds(r0, RSPAN)],
                        out_hbm.at[c].at[pl.ds(r0, RSPAN)])

    return _deg_kernel, _scatter_kernel


_RB_Y = 1000


def _y_body(d0_ref, d1_ref, x_ref, y_ref, yh_ref):
    deg = d0_ref[0, :, 0:1] + d1_ref[0, :, 0:1]
    dinv = lax.rsqrt(deg)
    y = x_ref[...] * dinv
    y_ref[...] = y
    yh_ref[...] = y * 0.5


_y_kernel = pl.pallas_call(
    _y_body,
    grid=(N // _RB_Y,),
    in_specs=[
        pl.BlockSpec((1, _RB_Y, DEGW), lambda i: (0, i, 0)),
        pl.BlockSpec((1, _RB_Y, DEGW), lambda i: (1, i, 0)),
        pl.BlockSpec((_RB_Y, D_IN), lambda i: (i, 0)),
    ],
    out_specs=[
        pl.BlockSpec((_RB_Y, D_IN), lambda i: (i, 0)),
        pl.BlockSpec((_RB_Y, D_IN), lambda i: (i, 0)),
    ],
    out_shape=[
        jax.ShapeDtypeStruct((N, D_IN), jnp.float32),
        jax.ShapeDtypeStruct((N, D_IN), jnp.float32),
    ],
)


_RB = 1000
_NB = N // _RB


def _final_body(d0_ref, d1_ref, s0_ref, s1_ref, batch_ref, w_ref, b_ref,
                h_ref, ge_ref, cnt_scr):
    i = pl.program_id(0)
    deg = d0_ref[0, :, 0:1] + d1_ref[0, :, 0:1]
    dinv = lax.rsqrt(deg)
    agg = (s0_ref[0] + s1_ref[0]) * dinv
    h = jnp.maximum(
        lax.dot_general(agg, w_ref[...], (((1,), (0,)), ((), ())),
                        precision=lax.Precision.HIGHEST,
                        preferred_element_type=jnp.float32) + b_ref[...],
        0.0)
    h_ref[...] = h

    oh = (lax.broadcasted_iota(jnp.int32, (G, _RB), 0)
          == batch_ref[0, 0:1, :]).astype(jnp.float32)
    part = lax.dot_general(oh, h, (((1,), (0,)), ((), ())),
                           precision=lax.Precision.HIGHEST,
                           preferred_element_type=jnp.float32)
    cpart = jnp.broadcast_to(jnp.sum(oh, axis=1, keepdims=True), (G, 128))

    @pl.when(i == 0)
    def _init():
        ge_ref[...] = jnp.zeros_like(ge_ref)
        cnt_scr[...] = jnp.zeros_like(cnt_scr)

    ge_ref[...] += part
    cnt_scr[...] += cpart

    @pl.when(i == _NB - 1)
    def _fin():
        ge_ref[...] = ge_ref[...] / jnp.maximum(cnt_scr[:, 0:1], 1.0)


_final_kernel = pl.pallas_call(
    _final_body,
    grid=(_NB,),
    in_specs=[
        pl.BlockSpec((1, _RB, DEGW), lambda i: (0, i, 0)),
        pl.BlockSpec((1, _RB, DEGW), lambda i: (1, i, 0)),
        pl.BlockSpec((1, _RB, D_IN), lambda i: (0, i, 0)),
        pl.BlockSpec((1, _RB, D_IN), lambda i: (1, i, 0)),
        pl.BlockSpec((1, 1, _RB), lambda i: (i, 0, 0)),
        pl.BlockSpec((D_IN, D_OUT), lambda i: (0, 0)),
        pl.BlockSpec((1, D_OUT), lambda i: (0, 0)),
    ],
    out_specs=[
        pl.BlockSpec((_RB, D_OUT), lambda i: (i, 0)),
        pl.BlockSpec((G, D_OUT), lambda i: (0, 0)),
    ],
    out_shape=[
        jax.ShapeDtypeStruct((N, D_OUT), jnp.float32),
        jax.ShapeDtypeStruct((G, D_OUT), jnp.float32),
    ],
    scratch_shapes=[pltpu.VMEM((G, 128), jnp.float32)],
)


def kernel(x, edge_index, batch, W, b):
    src3 = edge_index[0].reshape(NW, NCH, CH)
    dst3 = edge_index[1].reshape(NW, NCH, CH)
    deg_kernel, scatter_kernel = _sc_kernels()
    degp = deg_kernel(dst3)
    y, yh = _y_kernel(degp, degp, x)
    sp = scatter_kernel(src3, dst3, y, yh)
    h, ge = _final_kernel(degp, degp, sp, sp,
                          batch.reshape(_NB, 1, _RB), W, b.reshape(1, D_OUT))
    return (h, ge)
